# Initial kernel scaffold; baseline (speedup 1.0000x reference)
import jax
import jax.numpy as jnp
from jax.experimental import pallas as pl


def kernel(boxes, classification):
    raise NotImplementedError("write your pallas kernel here")


# speedup vs baseline: 19.7496x; 19.7496x over previous
import functools

import jax
import jax.numpy as jnp
from jax.experimental import pallas as pl
from jax.experimental.pallas import tpu as pltpu

_MAX_DET = 300
_SCORE_THR = 0.05
_IOU_THR = 0.5
_ROWS = 160
_LANES = 128
_NPAD = _ROWS * _LANES
_NEG = float("-inf")
_BIG = 2**30


def _nms_kernel(cls_ref, bx_ref, ob_ref, os_ref, ol_ref):
    C = cls_ref.shape[0]
    shp = (_ROWS, _LANES)

    def cls_body(c, carry):
        mx, am = carry
        x = cls_ref[c]
        upd = x > mx
        mx = jnp.where(upd, x, mx)
        am = jnp.where(upd, jnp.full(shp, c, jnp.int32), am)
        return mx, am

    scores, labels = jax.lax.fori_loop(
        0, C, cls_body,
        (jnp.full(shp, _NEG, jnp.float32), jnp.zeros(shp, jnp.int32)))

    x1 = bx_ref[0]
    y1 = bx_ref[1]
    x2 = bx_ref[2]
    y2 = bx_ref[3]
    areas = (x2 - x1) * (y2 - y1)

    row_i = jax.lax.broadcasted_iota(jnp.int32, shp, 0)
    lane_i = jax.lax.broadcasted_iota(jnp.int32, shp, 1)
    lin_i = row_i * _LANES + lane_i

    ob_ref[...] = jnp.full((_MAX_DET, 4), -1.0, jnp.float32)
    os_ref[...] = jnp.full((_MAX_DET, 1), -1.0, jnp.float32)
    ol_ref[...] = jnp.full((_MAX_DET, 1), -1, jnp.int32)

    cur0 = jnp.where(scores > _SCORE_THR, scores, _NEG)

    def body(t, cur):
        m = jnp.max(cur)
        any_valid = m > _NEG
        eq = cur == m
        lin = jnp.min(jnp.where(eq, lin_i, _BIG))
        sel = lin_i == lin
        fsel = sel.astype(jnp.float32)
        x1i = jnp.sum(fsel * x1)
        y1i = jnp.sum(fsel * y1)
        x2i = jnp.sum(fsel * x2)
        y2i = jnp.sum(fsel * y2)
        ai = jnp.sum(fsel * areas)
        li = jnp.sum(jnp.where(sel, labels, 0))

        xx1 = jnp.maximum(x1i, x1)
        yy1 = jnp.maximum(y1i, y1)
        xx2 = jnp.minimum(x2i, x2)
        yy2 = jnp.minimum(y2i, y2)
        inter = jnp.maximum(0.0, xx2 - xx1) * jnp.maximum(0.0, yy2 - yy1)
        iou = inter / (ai + areas - inter + 1e-8)
        cur = jnp.where((iou > _IOU_THR) | sel, _NEG, cur)

        @pl.when(any_valid)
        def _store():
            row = jnp.concatenate(
                [jnp.full((1, 1), v, jnp.float32)
                 for v in (x1i, y1i, x2i, y2i)], axis=1)
            ob_ref[pl.ds(t, 1), :] = row
            os_ref[pl.ds(t, 1), :] = jnp.full((1, 1), m, jnp.float32)
            ol_ref[pl.ds(t, 1), :] = jnp.full((1, 1), li, jnp.int32)

        return cur

    jax.lax.fori_loop(0, _MAX_DET, body, cur0)


@jax.jit
def kernel(boxes, classification):
    B, N, C = classification.shape
    pad = _NPAD - N
    cls_t = jnp.pad(classification, ((0, 0), (0, pad), (0, 0)),
                    constant_values=-1.0)
    cls_t = cls_t.transpose(0, 2, 1).reshape(B, C, _ROWS, _LANES)
    bx = jnp.pad(boxes, ((0, 0), (0, pad), (0, 0)))
    bx = bx.transpose(0, 2, 1).reshape(B, 4, _ROWS, _LANES)

    grid = (B,)
    ob, os_, ol = pl.pallas_call(
        _nms_kernel,
        grid=grid,
        in_specs=[
            pl.BlockSpec((None, C, _ROWS, _LANES), lambda b: (b, 0, 0, 0)),
            pl.BlockSpec((None, 4, _ROWS, _LANES), lambda b: (b, 0, 0, 0)),
        ],
        out_specs=[
            pl.BlockSpec((None, _MAX_DET, 4), lambda b: (b, 0, 0)),
            pl.BlockSpec((None, _MAX_DET, 1), lambda b: (b, 0, 0)),
            pl.BlockSpec((None, _MAX_DET, 1), lambda b: (b, 0, 0)),
        ],
        out_shape=[
            jax.ShapeDtypeStruct((B, _MAX_DET, 4), jnp.float32),
            jax.ShapeDtypeStruct((B, _MAX_DET, 1), jnp.float32),
            jax.ShapeDtypeStruct((B, _MAX_DET, 1), jnp.int32),
        ],
    )(cls_t, bx)
    return ob, os_.reshape(B, _MAX_DET), ol.reshape(B, _MAX_DET)
